# baseline (speedup 1.0000x reference)
import functools

import jax
import jax.numpy as jnp
from jax import lax
from jax.experimental import pallas as pl
from jax.experimental.pallas import tpu as pltpu
from jax.experimental.pallas import tpu_sc as plsc

VOCAB = 1000000
D = 64

_info = plsc.get_sparse_core_info()
NC, NS = _info.num_cores, _info.num_subcores
NW = NC * NS

B_TOTAL = 4096 * 50
CHUNK = 128
N_CHUNKS = B_TOTAL // CHUNK
CHUNKS_PER_W = N_CHUNKS // NW


def _gather_kernel(table_hbm, idx_hbm, out_hbm, idx_v, rows_v, sem):
    wid = lax.axis_index("s") * NC + lax.axis_index("c")
    base = wid * CHUNKS_PER_W
    pltpu.sync_copy(idx_hbm.at[wid], idx_v)

    def body(g, carry):
        pltpu.async_copy(table_hbm.at[idx_v.at[g]], rows_v, sem).wait()
        pltpu.sync_copy(rows_v, out_hbm.at[base + g])
        return carry

    lax.fori_loop(0, CHUNKS_PER_W, body, 0)


def _run(tokens_flat, embed_table):
    mesh = plsc.VectorSubcoreMesh(core_axis_name="c", subcore_axis_name="s")
    k = pl.kernel(
        _gather_kernel,
        mesh=mesh,
        out_type=jax.ShapeDtypeStruct((N_CHUNKS, CHUNK, D), jnp.float32),
        scratch_types=[
            pltpu.VMEM((CHUNKS_PER_W, CHUNK), jnp.int32),
            pltpu.VMEM((CHUNK, D), jnp.float32),
            pltpu.SemaphoreType.DMA,
        ],
        compiler_params=pltpu.CompilerParams(use_tc_tiling_on_sc=False),
    )
    return k(embed_table, tokens_flat)


def kernel(tokens_or_embeds, embed_table):
    B, T = tokens_or_embeds.shape
    idx = tokens_or_embeds.reshape(NW, CHUNKS_PER_W, CHUNK)
    out = _run(idx, embed_table)
    return out.reshape(B, T, D)


# speedup vs baseline: 1.0441x; 1.0441x over previous
import functools

import jax
import jax.numpy as jnp
from jax import lax
from jax.experimental import pallas as pl
from jax.experimental.pallas import tpu as pltpu
from jax.experimental.pallas import tpu_sc as plsc

VOCAB = 1000000
D = 64

_info = plsc.get_sparse_core_info()
NC, NS = _info.num_cores, _info.num_subcores
NW = NC * NS

B_TOTAL = 4096 * 50
CHUNK = 128
N_CHUNKS = B_TOTAL // CHUNK
CHUNKS_PER_W = N_CHUNKS // NW


NBUF = 10
OUTER = CHUNKS_PER_W // NBUF


def _gather_kernel(table_hbm, idx_hbm, out_hbm, idx_v, rows_v, gsem, ssem):
    wid = lax.axis_index("s") * NC + lax.axis_index("c")
    base = wid * CHUNKS_PER_W
    pltpu.sync_copy(idx_hbm.at[wid], idx_v)

    def body(o, carry):
        g0 = o * NBUF
        gd = [
            pltpu.async_copy(
                table_hbm.at[idx_v.at[g0 + b]], rows_v.at[b], gsem.at[b]
            )
            for b in range(NBUF)
        ]
        sd = []
        for b in range(NBUF):
            gd[b].wait()
            sd.append(
                pltpu.async_copy(rows_v.at[b], out_hbm.at[base + g0 + b], ssem.at[b])
            )
        for b in range(NBUF):
            sd[b].wait()
        return carry

    lax.fori_loop(0, OUTER, body, 0)


def _run(tokens_flat, embed_table):
    mesh = plsc.VectorSubcoreMesh(core_axis_name="c", subcore_axis_name="s")
    k = pl.kernel(
        _gather_kernel,
        mesh=mesh,
        out_type=jax.ShapeDtypeStruct((N_CHUNKS, CHUNK, D), jnp.float32),
        scratch_types=[
            pltpu.VMEM((CHUNKS_PER_W, CHUNK), jnp.int32),
            pltpu.VMEM((NBUF, CHUNK, D), jnp.float32),
            pltpu.SemaphoreType.DMA((NBUF,)),
            pltpu.SemaphoreType.DMA((NBUF,)),
        ],
        compiler_params=pltpu.CompilerParams(use_tc_tiling_on_sc=False),
    )
    return k(embed_table, tokens_flat)


def kernel(tokens_or_embeds, embed_table):
    B, T = tokens_or_embeds.shape
    idx = tokens_or_embeds.reshape(NW, CHUNKS_PER_W, CHUNK)
    out = _run(idx, embed_table)
    return out.reshape(B, T, D)
